# Initial kernel scaffold; baseline (speedup 1.0000x reference)
import jax
import jax.numpy as jnp
from jax.experimental import pallas as pl


def kernel(x, expert_emb, exp_fc1_w, exp_fc1_b, exp_fc2_w, exp_fc2_b, sw_fc1_w, sw_fc1_b, sw_fc2_w, sw_fc2_b, ch_fc1_w, ch_fc1_b, ch_fc2_w, ch_fc2_b):
    raise NotImplementedError("write your pallas kernel here")


# speedup vs baseline: 1.0131x; 1.0131x over previous
import jax
import jax.numpy as jnp
from jax.experimental import pallas as pl
from jax.experimental.pallas import tpu as pltpu

_HI = jax.lax.Precision.HIGHEST


def _gelu(v):
    return 0.5 * v * (1.0 + jax.lax.erf(v * 0.7071067811865475))


def _router_kernel(x_ref, emb_ref, sel_ref, *, bsz, ntok, dim, nexp):
    T = bsz * ntok
    x2 = x_ref[:]
    logits = jnp.dot(x2.astype(jnp.bfloat16), emb_ref[:].astype(jnp.bfloat16).T,
                     preferred_element_type=jnp.float32)
    col = jax.lax.broadcasted_iota(jnp.int32, (T, nexp), 1)
    m1 = jnp.max(logits, axis=1, keepdims=True)
    i1 = jnp.min(jnp.where(logits == m1, col, nexp), axis=1, keepdims=True)
    masked = jnp.where(col == i1, -jnp.inf, logits)
    m2 = jnp.max(masked, axis=1, keepdims=True)
    i2 = jnp.min(jnp.where(masked == m2, col, nexp), axis=1, keepdims=True)
    t = jax.lax.broadcasted_iota(jnp.int32, (T, 1), 0)
    base = t - t % ntok
    src = jnp.concatenate([(base + i1).astype(jnp.float32),
                           (base + i2).astype(jnp.float32)], axis=1)
    q = (t % bsz) * ntok + t // bsz
    colT = jax.lax.broadcasted_iota(jnp.int32, (T, T), 1)
    perm = (colT == q).astype(jnp.float32)
    srcp = jnp.dot(perm, src, preferred_element_type=jnp.float32,
                   precision=_HI)
    s1 = srcp[:, 0:1].astype(jnp.int32)
    s2 = srcp[:, 1:2].astype(jnp.int32)
    oh1 = (colT == s1).astype(jnp.float32)
    oh2 = (colT == s2).astype(jnp.float32)
    g1 = jnp.dot(oh1, x2, preferred_element_type=jnp.float32, precision=_HI)
    g2 = jnp.dot(oh2, x2, preferred_element_type=jnp.float32, precision=_HI)
    sel = jnp.concatenate([g1, g2], axis=1)
    sel_ref[:] = sel.reshape(nexp, bsz, 2 * dim)


def _sw_kernel(x_ref, w1_ref, b1_ref, w2_ref, b2_ref, wts_ref, acc_ref):
    s = pl.program_id(0)
    xb = x_ref[:].astype(jnp.bfloat16)
    wb = w1_ref[:].astype(jnp.bfloat16)
    part = jnp.dot(xb, wb.T, preferred_element_type=jnp.float32)
    h = _gelu(part + b1_ref[:])
    w2b = w2_ref[:].astype(jnp.bfloat16)
    contrib = jnp.dot(h.astype(jnp.bfloat16), w2b.T,
                      preferred_element_type=jnp.float32)

    @pl.when(s == 0)
    def _():
        acc_ref[:] = contrib

    @pl.when(s > 0)
    def _():
        acc_ref[:] = acc_ref[:] + contrib

    @pl.when(s == pl.num_programs(0) - 1)
    def _():
        logits = acc_ref[:] + b2_ref[:]
        m = jnp.max(logits, axis=1, keepdims=True)
        ez = jnp.exp(logits - m)
        wts_ref[:] = ez / jnp.sum(ez, axis=1, keepdims=True)


def _fc1_kernel(sel_ref, w_ref, b_ref, h_ref):
    sb = sel_ref[0].astype(jnp.bfloat16)
    wb = w_ref[0].astype(jnp.bfloat16)
    h = jnp.dot(sb, wb.T, preferred_element_type=jnp.float32) + b_ref[0]
    h_ref[0] = _gelu(h)


def _fc2_kernel(h_ref, w_ref, b_ref, wts_ref, out_ref, *, nexp):
    e = pl.program_id(0)
    hb = h_ref[0].astype(jnp.bfloat16)
    wb = w_ref[0].astype(jnp.bfloat16)
    r = jnp.dot(hb, wb.T, preferred_element_type=jnp.float32) + b_ref[0]
    ecol = jax.lax.broadcasted_iota(jnp.int32, (nexp, 1), 0)
    onehot = (ecol == e).astype(jnp.float32)
    wcol = jnp.dot(wts_ref[:], onehot, preferred_element_type=jnp.float32,
                   precision=_HI)
    contrib = r * wcol

    @pl.when(e == 0)
    def _():
        out_ref[:] = contrib

    @pl.when(e > 0)
    def _():
        out_ref[:] = out_ref[:] + contrib


def _head_kernel(ws_ref, w1_ref, b1_ref, w2_ref, b2_ref, out_ref):
    wsb = ws_ref[:].astype(jnp.bfloat16)
    h = jnp.dot(wsb, w1_ref[:].astype(jnp.bfloat16).T,
                preferred_element_type=jnp.float32) + b1_ref[:]
    hb = _gelu(h).astype(jnp.bfloat16)
    out_ref[:] = jnp.dot(hb, w2_ref[:].astype(jnp.bfloat16).T,
                         preferred_element_type=jnp.float32) + b2_ref[:]


def kernel(x, expert_emb, exp_fc1_w, exp_fc1_b, exp_fc2_w, exp_fc2_b,
           sw_fc1_w, sw_fc1_b, sw_fc2_w, sw_fc2_b,
           ch_fc1_w, ch_fc1_b, ch_fc2_w, ch_fc2_b):
    import functools
    bsz, ntok, dim = x.shape
    nexp = expert_emb.shape[0]
    ed = exp_fc1_w.shape[1]
    ncls = ch_fc2_w.shape[0]
    f32 = jnp.float32

    x_tok = x.reshape(bsz * ntok, dim)
    x_flat = x.reshape(bsz, ntok * dim)

    sel = pl.pallas_call(
        functools.partial(_router_kernel, bsz=bsz, ntok=ntok, dim=dim,
                          nexp=nexp),
        out_shape=jax.ShapeDtypeStruct((nexp, bsz, ed), f32),
    )(x_tok, expert_emb)

    SWB = 512
    nsteps = (ntok * dim) // SWB
    wts = pl.pallas_call(
        _sw_kernel,
        grid=(nsteps,),
        in_specs=[
            pl.BlockSpec((bsz, ntok * dim), lambda s: (0, 0)),
            pl.BlockSpec((SWB, ntok * dim), lambda s: (s, 0)),
            pl.BlockSpec((1, SWB), lambda s: (0, s)),
            pl.BlockSpec((nexp, SWB), lambda s: (0, s)),
            pl.BlockSpec((1, nexp), lambda s: (0, 0)),
        ],
        out_specs=pl.BlockSpec((bsz, nexp), lambda s: (0, 0)),
        out_shape=jax.ShapeDtypeStruct((bsz, nexp), f32),
        scratch_shapes=[pltpu.VMEM((bsz, nexp), f32)],
        compiler_params=pltpu.CompilerParams(
            vmem_limit_bytes=60 * 1024 * 1024),
    )(x_flat, sw_fc1_w, sw_fc1_b.reshape(1, -1), sw_fc2_w,
      sw_fc2_b.reshape(1, -1))

    h1 = pl.pallas_call(
        _fc1_kernel,
        grid=(nexp,),
        in_specs=[
            pl.BlockSpec((1, bsz, ed), lambda e: (e, 0, 0)),
            pl.BlockSpec((1, ed, ed), lambda e: (e, 0, 0)),
            pl.BlockSpec((1, 1, ed), lambda e: (e, 0, 0)),
        ],
        out_specs=pl.BlockSpec((1, bsz, ed), lambda e: (e, 0, 0)),
        out_shape=jax.ShapeDtypeStruct((nexp, bsz, ed), f32),
        compiler_params=pltpu.CompilerParams(
            vmem_limit_bytes=60 * 1024 * 1024),
    )(sel, exp_fc1_w, exp_fc1_b.reshape(nexp, 1, ed))

    ws = pl.pallas_call(
        functools.partial(_fc2_kernel, nexp=nexp),
        grid=(nexp,),
        in_specs=[
            pl.BlockSpec((1, bsz, ed), lambda e: (e, 0, 0)),
            pl.BlockSpec((1, ed, ed), lambda e: (e, 0, 0)),
            pl.BlockSpec((1, 1, ed), lambda e: (e, 0, 0)),
            pl.BlockSpec((bsz, nexp), lambda e: (0, 0)),
        ],
        out_specs=pl.BlockSpec((bsz, ed), lambda e: (0, 0)),
        out_shape=jax.ShapeDtypeStruct((bsz, ed), f32),
        compiler_params=pltpu.CompilerParams(
            vmem_limit_bytes=60 * 1024 * 1024),
    )(h1, exp_fc2_w, exp_fc2_b.reshape(nexp, 1, ed), wts)

    out = pl.pallas_call(
        _head_kernel,
        in_specs=[
            pl.BlockSpec((bsz, ed), lambda: (0, 0)),
            pl.BlockSpec((ed, ed), lambda: (0, 0)),
            pl.BlockSpec((1, ed), lambda: (0, 0)),
            pl.BlockSpec((ncls, ed), lambda: (0, 0)),
            pl.BlockSpec((1, ncls), lambda: (0, 0)),
        ],
        out_specs=pl.BlockSpec((bsz, ncls), lambda: (0, 0)),
        out_shape=jax.ShapeDtypeStruct((bsz, ncls), f32),
        compiler_params=pltpu.CompilerParams(
            vmem_limit_bytes=60 * 1024 * 1024),
    )(ws, ch_fc1_w, ch_fc1_b.reshape(1, -1), ch_fc2_w,
      ch_fc2_b.reshape(1, -1))
    return out
